# baseline (speedup 1.0000x reference)
import functools
import jax
import jax.numpy as jnp
from jax import lax
from jax.experimental import pallas as pl
from jax.experimental.pallas import tpu as pltpu
from jax.experimental.pallas import tpu_sc as plsc

D_MODEL = 64
BATCH = 4096
SEQ = 200
SCALE = 8.0

NC = 2
NS = 16
NW = NC * NS
LANES = 16

VPAIR = 500000
SBLK = SEQ // 8
CBLK = BATCH // 128
G = 4
O = 2

_mesh = plsc.VectorSubcoreMesh(core_axis_name="c", subcore_axis_name="s")


@functools.partial(
    pl.kernel,
    out_type=jax.ShapeDtypeStruct((SEQ, D_MODEL, BATCH), jnp.float32),
    mesh=_mesh,
    scratch_types=[
        pltpu.VMEM((SBLK, 8, 128), jnp.int32),
        pltpu.VMEM((G, 128), jnp.int32),
        pltpu.VMEM((G, 128, 128), jnp.float32),
        pltpu.VMEM((O, D_MODEL, 128), jnp.float32),
        pltpu.SemaphoreType.DMA((G,)),
        pltpu.SemaphoreType.DMA((O,)),
        pltpu.SemaphoreType.DMA,
    ],
    compiler_params=pltpu.CompilerParams(use_tc_tiling_on_sc=True, needs_layout_passes=False),
)
def _emb_lookup(xt_hbm, lut_hbm, out_hbm, xb, ih, g, ob, gsem, wsem, xsem):
    c = lax.axis_index("s") * NC + lax.axis_index("c")
    iota = lax.iota(jnp.int32, LANES)
    rowv = [iota + bg * LANES for bg in range(8)]

    for s8 in range(SBLK):
        pltpu.async_copy(
            xt_hbm.at[pl.ds(s8 * 8, 8), pl.ds(c * 128, 128)], xb.at[s8], xsem)
    for s8 in range(SBLK):
        pltpu.make_async_copy(
            xt_hbm.at[pl.ds(0, 8), pl.ds(0, 128)], xb.at[s8], xsem).wait()

    def compute_ih(r, s):
        s8, sl = s // 8, s % 8
        for bg in range(8):
            xv = xb[s8, sl, pl.ds(bg * LANES, LANES)]
            ih[r, pl.ds(bg * LANES, LANES)] = lax.shift_right_logical(xv, 1)

    def start_gather(r):
        pltpu.async_copy(lut_hbm.at[ih.at[r]], g.at[r], gsem.at[r])

    def wait_gather(r):
        pltpu.make_async_copy(lut_hbm.at[ih.at[r]], g.at[r], gsem.at[r]).wait()

    def extract(r, o, s):
        s8, sl = s // 8, s % 8
        hvs = []
        for bg in range(8):
            xv = xb[s8, sl, pl.ds(bg * LANES, LANES)]
            hvs.append(lax.shift_left(lax.bitwise_and(xv, 1), 6))
        gr = g.at[r]
        obr = ob.at[o]

        @plsc.parallel_loop(0, D_MODEL, unroll=8)
        def _(d):
            for bg in range(8):
                vals = plsc.load_gather(gr, [rowv[bg], hvs[bg] + d])
                obr[d, pl.ds(bg * LANES, LANES)] = vals * SCALE

    def start_write(o, s):
        pltpu.async_copy(ob.at[o], out_hbm.at[s, :, pl.ds(c * 128, 128)],
                         wsem.at[o])

    def wait_write(o):
        pltpu.make_async_copy(ob.at[o], out_hbm.at[0, :, pl.ds(0, 128)],
                              wsem.at[o]).wait()

    for s in range(G):
        compute_ih(s % G, s)
        start_gather(s % G)

    for b in range(G):
        s = b
        r, o = s % G, s % O
        wait_gather(r)
        if s >= O:
            wait_write(o)
        extract(r, o, s)
        start_write(o, s)
        compute_ih(r, s + G)
        start_gather(r)

    @pl.loop(1, SEQ // G - 1)
    def _(grp):
        s0 = grp * G
        for b in range(G):
            s = s0 + b
            r, o = b % G, b % O
            wait_gather(r)
            wait_write(o)
            extract(r, o, s)
            start_write(o, s)
            compute_ih(r, s + G)
            start_gather(r)

    for b in range(G):
        s = SEQ - G + b
        r, o = s % G, s % O
        wait_gather(r)
        wait_write(o)
        extract(r, o, s)
        start_write(o, s)

    for o in range(O):
        wait_write(o)


def kernel(x, lut):
    xt = x.T
    lut2 = lut.reshape(VPAIR, 128)
    out = _emb_lookup(xt, lut2)
    return out.transpose(2, 0, 1)


# speedup vs baseline: 1.5361x; 1.5361x over previous
import functools
import jax
import jax.numpy as jnp
from jax import lax
from jax.experimental import pallas as pl
from jax.experimental.pallas import tpu as pltpu
from jax.experimental.pallas import tpu_sc as plsc

D_MODEL = 64
BATCH = 4096
SEQ = 200
SCALE = 8.0

NC = 2
NS = 16
NW = NC * NS
LANES = 16

VPAIR = 500000
SBLK = SEQ // 8
CBLK = BATCH // 128
G = 4
O = 2

_mesh = plsc.VectorSubcoreMesh(core_axis_name="c", subcore_axis_name="s")


@functools.partial(
    pl.kernel,
    out_type=jax.ShapeDtypeStruct((SEQ, D_MODEL, BATCH), jnp.float32),
    mesh=_mesh,
    scratch_types=[
        pltpu.VMEM((SBLK, 8, 128), jnp.int32),
        pltpu.VMEM((G, 128), jnp.int32),
        pltpu.VMEM((G, 128, 128), jnp.float32),
        pltpu.VMEM((O, D_MODEL, 128), jnp.float32),
        pltpu.SemaphoreType.DMA((G,)),
        pltpu.SemaphoreType.DMA((O,)),
        pltpu.SemaphoreType.DMA,
    ],
    compiler_params=pltpu.CompilerParams(use_tc_tiling_on_sc=True, needs_layout_passes=False),
)
def _emb_lookup(xt_hbm, lut_hbm, out_hbm, xb, ih, g, ob, gsem, wsem, xsem):
    c = lax.axis_index("s") * NC + lax.axis_index("c")
    iota = lax.iota(jnp.int32, LANES)
    rowv = [iota + bg * LANES for bg in range(8)]

    for s8 in range(SBLK):
        pltpu.async_copy(
            xt_hbm.at[pl.ds(s8 * 8, 8), pl.ds(c * 128, 128)], xb.at[s8], xsem)
    for s8 in range(SBLK):
        pltpu.make_async_copy(
            xt_hbm.at[pl.ds(0, 8), pl.ds(0, 128)], xb.at[s8], xsem).wait()

    def compute_ih(r, s):
        s8, sl = s // 8, s % 8
        for bg in range(8):
            xv = xb[s8, sl, pl.ds(bg * LANES, LANES)]
            ih[r, pl.ds(bg * LANES, LANES)] = lax.shift_right_logical(xv, 1)

    def start_gather(r):
        pltpu.async_copy(lut_hbm.at[ih.at[r]], g.at[r], gsem.at[r])

    def wait_gather(r):
        pltpu.make_async_copy(lut_hbm.at[ih.at[r]], g.at[r], gsem.at[r]).wait()

    def extract(r, o, s):
        s8, sl = s // 8, s % 8
        hvs = []
        for bg in range(8):
            xv = xb[s8, sl, pl.ds(bg * LANES, LANES)]
            hvs.append(lax.shift_left(lax.bitwise_and(xv, 1), 6))
        gr = g.at[r]
        obr = ob.at[o]

        @plsc.parallel_loop(0, D_MODEL, unroll=4)
        def _(d):
            dv = lax.bitwise_and(d + iota, D_MODEL - 1)
            for bg in range(8):
                vals = plsc.load_gather(gr, [rowv[bg], hvs[bg] + dv])
                plsc.store_scatter(obr, [dv, rowv[bg]], vals * SCALE)

    def start_write(o, s):
        pltpu.async_copy(ob.at[o], out_hbm.at[s, :, pl.ds(c * 128, 128)],
                         wsem.at[o])

    def wait_write(o):
        pltpu.make_async_copy(ob.at[o], out_hbm.at[0, :, pl.ds(0, 128)],
                              wsem.at[o]).wait()

    for s in range(G):
        compute_ih(s % G, s)
        start_gather(s % G)

    for b in range(G):
        s = b
        r, o = s % G, s % O
        wait_gather(r)
        if s >= O:
            wait_write(o)
        extract(r, o, s)
        start_write(o, s)
        compute_ih(r, s + G)
        start_gather(r)

    @pl.loop(1, SEQ // G - 1)
    def _(grp):
        s0 = grp * G
        for b in range(G):
            s = s0 + b
            r, o = b % G, b % O
            wait_gather(r)
            wait_write(o)
            extract(r, o, s)
            start_write(o, s)
            compute_ih(r, s + G)
            start_gather(r)

    for b in range(G):
        s = SEQ - G + b
        r, o = s % G, s % O
        wait_gather(r)
        wait_write(o)
        extract(r, o, s)
        start_write(o, s)

    for o in range(O):
        wait_write(o)


def kernel(x, lut):
    xt = x.T
    lut2 = lut.reshape(VPAIR, 128)
    out = _emb_lookup(xt, lut2)
    return out.transpose(2, 0, 1)


# speedup vs baseline: 2.4259x; 1.5793x over previous
import functools
import jax
import jax.numpy as jnp
from jax import lax
from jax.experimental import pallas as pl
from jax.experimental.pallas import tpu as pltpu
from jax.experimental.pallas import tpu_sc as plsc

D_MODEL = 64
BATCH = 4096
SEQ = 200
SCALE = 8.0

NC = 2
NS = 16
NW = NC * NS
LANES = 16

VBIG = 1000000
NFB = 7812
SBLK = SEQ // 8
CBLK = BATCH // 128
G = 4
O = 2

_mesh = plsc.VectorSubcoreMesh(core_axis_name="c", subcore_axis_name="s")


@functools.partial(
    pl.kernel,
    out_type=jax.ShapeDtypeStruct((SEQ, D_MODEL, BATCH), jnp.float32),
    mesh=_mesh,
    scratch_types=[
        pltpu.VMEM((SBLK, 8, 128), jnp.int32),
        pltpu.VMEM((G, 128), jnp.int32),
        pltpu.VMEM((G, 128, 128), jnp.float32),
        pltpu.VMEM((O, D_MODEL, 128), jnp.float32),
        pltpu.SemaphoreType.DMA((G,)),
        pltpu.SemaphoreType.DMA((O,)),
        pltpu.SemaphoreType.DMA,
    ],
    compiler_params=pltpu.CompilerParams(use_tc_tiling_on_sc=True, needs_layout_passes=False),
)
def _emb_lookup(xt_hbm, lut_hbm, out_hbm, xb, ih, g, ob, gsem, wsem, xsem):
    c = lax.axis_index("s") * NC + lax.axis_index("c")
    iota = lax.iota(jnp.int32, LANES)
    rowv = [iota + bg * LANES for bg in range(8)]

    for s8 in range(SBLK):
        pltpu.async_copy(
            xt_hbm.at[pl.ds(s8 * 8, 8), pl.ds(c * 128, 128)], xb.at[s8], xsem)
    for s8 in range(SBLK):
        pltpu.make_async_copy(
            xt_hbm.at[pl.ds(0, 8), pl.ds(0, 128)], xb.at[s8], xsem).wait()

    def compute_ih(r, s):
        s8, sl = s // 8, s % 8
        for bg in range(8):
            ih[r, pl.ds(bg * LANES, LANES)] = xb[s8, sl, pl.ds(bg * LANES, LANES)]

    def start_gather(r):
        pltpu.async_copy(lut_hbm.at[ih.at[r]], g.at[r], gsem.at[r])

    def wait_gather(r):
        pltpu.make_async_copy(lut_hbm.at[ih.at[r]], g.at[r], gsem.at[r]).wait()

    def extract(r, o, s):
        gr = g.at[r]
        obr = ob.at[o]

        @plsc.parallel_loop(0, D_MODEL, unroll=4)
        def _(d):
            dv = lax.bitwise_and(d + iota, D_MODEL - 1)
            for bg in range(8):
                vals = plsc.load_gather(gr, [rowv[bg], dv])
                plsc.store_scatter(obr, [dv, rowv[bg]], vals * SCALE)

    def start_write(o, s):
        pltpu.async_copy(ob.at[o], out_hbm.at[s, :, pl.ds(c * 128, 128)],
                         wsem.at[o])

    def wait_write(o):
        pltpu.make_async_copy(ob.at[o], out_hbm.at[0, :, pl.ds(0, 128)],
                              wsem.at[o]).wait()

    for s in range(G):
        compute_ih(s % G, s)
        start_gather(s % G)

    for b in range(G):
        s = b
        r, o = s % G, s % O
        wait_gather(r)
        if s >= O:
            wait_write(o)
        extract(r, o, s)
        start_write(o, s)
        compute_ih(r, s + G)
        start_gather(r)

    @pl.loop(1, SEQ // G - 1)
    def _(grp):
        s0 = grp * G
        for b in range(G):
            s = s0 + b
            r, o = b % G, b % O
            wait_gather(r)
            wait_write(o)
            extract(r, o, s)
            start_write(o, s)
            compute_ih(r, s + G)
            start_gather(r)

    for b in range(G):
        s = SEQ - G + b
        r, o = s % G, s % O
        wait_gather(r)
        wait_write(o)
        extract(r, o, s)
        start_write(o, s)

    for o in range(O):
        wait_write(o)


@functools.partial(
    pl.kernel,
    out_type=jax.ShapeDtypeStruct((VBIG, 128), jnp.float32),
    mesh=_mesh,
    scratch_types=[
        pltpu.VMEM((2, D_MODEL, 128), jnp.float32),
        pltpu.VMEM((2, 128, 128), jnp.float32),
        pltpu.VMEM((D_MODEL, 128), jnp.float32),
        pltpu.SemaphoreType.DMA((2,)),
        pltpu.SemaphoreType.DMA((2,)),
    ],
    compiler_params=pltpu.CompilerParams(use_tc_tiling_on_sc=True,
                                         needs_layout_passes=False),
)
def _transpose(lutT_hbm, tail_hbm, big_hbm, srcv, dstv, tailv, rsem, wsem):
    w = lax.axis_index("s") * NC + lax.axis_index("c")
    iota = lax.iota(jnp.int32, LANES)
    dvs = [bg * LANES + iota for bg in range(4)]

    def kof(j):
        return w + j * NW

    def start_read(r, k):
        pltpu.async_copy(lutT_hbm.at[:, pl.ds(k * 128, 128)], srcv.at[r],
                         rsem.at[r])

    def wait_read(r):
        pltpu.make_async_copy(lutT_hbm.at[:, pl.ds(0, 128)], srcv.at[r],
                              rsem.at[r]).wait()

    def start_write(r, k):
        pltpu.async_copy(dstv.at[r], big_hbm.at[pl.ds(k * 128, 128), :],
                         wsem.at[r])

    def wait_write(r):
        pltpu.make_async_copy(dstv.at[r], big_hbm.at[pl.ds(0, 128), :],
                              wsem.at[r]).wait()

    def transpose_block(r):
        sr = srcv.at[r]
        dr = dstv.at[r]

        @plsc.parallel_loop(0, 128, unroll=4)
        def _(m):
            p = lax.shift_right_logical(m, 3)
            rr0 = lax.bitwise_and(m, 7)
            rrv = rr0 * LANES + lax.bitwise_and(iota + p, LANES - 1)
            for bg in range(4):
                vals = plsc.load_gather(sr, [dvs[bg], rrv])
                plsc.store_scatter(dr, [rrv, dvs[bg]], vals)

    @pl.when(w == NW - 1)
    def _():
        pltpu.sync_copy(tail_hbm, tailv)
        pltpu.sync_copy(tailv, big_hbm.at[pl.ds(NFB * 128, D_MODEL), :])

    start_read(0, kof(0))
    start_read(1, kof(1))

    for half in range(2):
        wait_read(half)
        transpose_block(half)
        start_write(half, kof(half))
        start_read(half, kof(half + 2))

    @pl.loop(1, 122)
    def _(j2):
        for half in range(2):
            j = j2 * 2 + half
            r = half
            wait_read(r)
            wait_write(r)
            transpose_block(r)
            start_write(r, kof(j))
            kpf = jnp.minimum(kof(j + 2), NFB - 1)
            start_read(r, kpf)

    wait_read(0)
    wait_read(1)
    wait_write(0)

    @pl.when(w < NFB - (NFB // NW) * NW)
    def _():
        transpose_block(0)
        start_write(0, kof(244))

    wait_write(1)

    @pl.when(w < NFB - (NFB // NW) * NW)
    def _():
        wait_write(0)


def kernel(x, lut):
    xt = x.T
    lutT = lut.T
    tail = jnp.pad(lut[NFB * 128:], ((0, 0), (0, D_MODEL)))
    big = _transpose(lutT, tail)
    out = _emb_lookup(xt, big)
    return out.transpose(2, 0, 1)
